# baseline (speedup 1.0000x reference)
import jax
import jax.numpy as jnp
from jax import lax
from jax.experimental import pallas as pl
from jax.experimental.pallas import tpu as pltpu
from jax.experimental.pallas import tpu_sc as plsc

B = 16
T = 1024
L = 16
NCHUNK = T // L
NEG_INF = float("-inf")


def _sc_body(starts_hbm, ends_hbm, frames_hbm, out_hbm, vs, ve, vf, vout):
    b = lax.axis_index("s")
    pltpu.sync_copy(starts_hbm.at[b], vs)
    pltpu.sync_copy(ends_hbm.at[b], ve)
    pltpu.sync_copy(frames_hbm.at[b], vf)

    lane = lax.broadcasted_iota(jnp.int32, (L,), 0)
    lshift = jnp.maximum(lane - 1, 0)

    def load16(ref, i):
        return ref[lax.shift_right_logical(i, 3),
                   pl.ds(jnp.bitwise_and(i, 7) * L, L)]

    UNROLL = 4

    def phase_a(j, carry):
        rv, bestv, bestc = carry
        for k in range(UNROLL):
            i = j * UNROLL + k
            s_vec = load16(vs, i)
            e_vec = load16(ve, i)
            incl = plsc.cummax(s_vec)
            ex = incl.at[lshift].get(mode="promise_in_bounds")
            pfx = jnp.maximum(jnp.where(lane == 0, NEG_INF, ex), rv)
            cand = pfx + e_vec
            upd = cand > bestv
            bestc = jnp.where(upd, i, bestc)
            bestv = jnp.where(upd, cand, bestv)
            rv = jnp.maximum(rv, jnp.max(s_vec))
        return rv, bestv, bestc

    init_a = (jnp.full((L,), NEG_INF, jnp.float32),
              jnp.full((L,), NEG_INF, jnp.float32),
              jnp.zeros((L,), jnp.int32))
    _, bestv, bestc = lax.fori_loop(0, NCHUNK // UNROLL, phase_a, init_a)

    gmax = jnp.max(bestv)
    e_cand = bestc * L + lane
    e_star = jnp.min(jnp.where(bestv == gmax, e_cand, T))
    chunk_e = lax.shift_right_logical(e_star, 4)
    lane_e = jnp.bitwise_and(e_star, L - 1)

    def phase_b(j, carry):
        sbv, sbc = carry
        for k in range(UNROLL):
            i = j * UNROLL + k
            s_vec = load16(vs, i)
            valid = jnp.logical_or(
                i < chunk_e,
                jnp.logical_and(i == chunk_e, lane < lane_e))
            sv = jnp.where(valid, s_vec, NEG_INF)
            upd = sv > sbv
            sbc = jnp.where(upd, i, sbc)
            sbv = jnp.where(upd, sv, sbv)
        return sbv, sbc

    init_b = (jnp.full((L,), NEG_INF, jnp.float32),
              jnp.zeros((L,), jnp.int32))
    trips_b = lax.shift_right_logical(chunk_e + UNROLL, 2)
    sbv, sbc = lax.fori_loop(0, trips_b, phase_b, init_b)
    smax = jnp.max(sbv)
    s_star = jnp.min(jnp.where(sbv == smax, sbc * L + lane, T))

    idx = jnp.where(lane == 0, s_star, e_star)
    frames = plsc.load_gather(vf, [lax.shift_right_logical(idx, 7),
                                   jnp.bitwise_and(idx, 127)])
    res = frames.astype(jnp.float32) + jnp.where(
        lane == 1, jnp.float32(1.0), jnp.float32(0.0))
    vout[...] = res
    pltpu.sync_copy(vout, out_hbm.at[b, 0, pl.ds(0, L)])


@jax.jit
def _post_process(starts, ends, frames):
    mesh = plsc.VectorSubcoreMesh(
        core_axis_name="c", subcore_axis_name="s", num_cores=1)
    run = pl.kernel(
        _sc_body,
        out_type=jax.ShapeDtypeStruct((B, 8, 128), jnp.float32),
        mesh=mesh,
        compiler_params=pltpu.CompilerParams(needs_layout_passes=False),
        scratch_types=[
            pltpu.VMEM((8, 128), jnp.float32),
            pltpu.VMEM((8, 128), jnp.float32),
            pltpu.VMEM((8, 128), jnp.int32),
            pltpu.VMEM((L,), jnp.float32),
        ],
    )
    out = run(starts, ends, frames)
    return out[:, 0, :2]


def kernel(temporal_dist, time_mask, frames_id):
    del time_mask
    starts = temporal_dist[:, :, 0].reshape(B, 8, 128)
    ends = temporal_dist[:, :, 1].reshape(B, 8, 128)
    frames = frames_id.astype(jnp.int32).reshape(B, 8, 128)
    return _post_process(starts, ends, frames)
